# baseline (speedup 1.0000x reference)
import functools

import jax
import jax.numpy as jnp
from jax import lax
from jax.experimental import pallas as pl
from jax.experimental.pallas import tpu as pltpu
from jax.experimental.pallas import tpu_sc as plsc

N = 1024
K = 102
NB = 1024
NW = 32
RPW = N // NW
LANES = 16
NPOS = NB // LANES


def _prep_kernel(emb_ref, u_ref, inv_ref, pos_ref):
    e = emb_ref[...]
    eb = e.astype(jnp.bfloat16)
    g = lax.dot_general(
        eb, eb, (((1,), (1,)), ((), ())), preferred_element_type=jnp.float32
    )
    rows = lax.broadcasted_iota(jnp.int32, (N, N), 0)
    cols = lax.broadcasted_iota(jnp.int32, (N, N), 1)
    diag = jnp.sum(jnp.where(rows == cols, g, 0.0), axis=1, keepdims=True)
    inv = 1.0 / jnp.maximum(jnp.sqrt(diag), 1e-12)
    u = g * inv.reshape(1, N)
    shift = jnp.where(rows < N // 2, rows + N // 2, rows - N // 2)
    pos = inv * jnp.sum(jnp.where(cols == shift, u, 0.0), axis=1, keepdims=True)
    u_ref[...] = u
    inv_ref[...] = inv
    pos_ref[...] = pos


def _sc_body(u_hbm, inv_hbm, dpart_hbm, ublk, invb, cnt, emid, dacc, sem):
    cid = lax.axis_index("c")
    sid = lax.axis_index("s")
    wid = sid * 2 + cid
    base = wid * RPW
    ucopy = pltpu.make_async_copy(u_hbm.at[pl.ds(base, RPW)], ublk, sem)
    ucopy.start()
    pltpu.sync_copy(inv_hbm.at[pl.ds(base, RPW)], invb)

    lane_f = lax.iota(jnp.int32, LANES).astype(jnp.float32)
    ones = jnp.full((LANES,), 1.0, dtype=jnp.float32)
    zeros = jnp.zeros((LANES,), dtype=jnp.float32)

    def fill_emid(p, _):
        arg = lane_f * (4.0 * NPOS / NB) + (
            p.astype(jnp.float32) * (4.0 / NB) + (2.0 / NB - 2.0)
        )
        emid[p, :] = jnp.exp(arg)
        return 0

    lax.fori_loop(0, NPOS, fill_emid, 0)

    @plsc.parallel_loop(0, RPW * NPOS, unroll=16)
    def _zero(i):
        cnt[i, :] = zeros

    ucopy.wait()

    def data_row(r, _):
        inv_r = plsc.load_gather(invb, [jnp.full((LANES,), r, dtype=jnp.int32)])
        scale = inv_r * (NB / 2.0)
        fbase = r * NPOS

        @plsc.parallel_loop(0, N // LANES, unroll=8)
        def _chunk(c):
            v = ublk[r, pl.ds(c * LANES, LANES)]
            bf = v * scale + (NB / 2.0)
            b = bf.astype(jnp.int32)
            b = jnp.minimum(jnp.maximum(b, 0), NB - 1)
            fi = fbase + (b & (NPOS - 1))
            ln = b >> 6
            plsc.addupdate_scatter(cnt, [fi, ln], ones)

        return 0

    lax.fori_loop(0, RPW, data_row, 0)

    def scan_row(r, _):
        fbase = r * NPOS

        def acc_tot(c, t):
            return t + cnt[fbase + c, :]

        tot = lax.fori_loop(0, NPOS, acc_tot, zeros, unroll=8)
        pfx = plsc.cumsum(tot) - tot

        def acc_d(c2, carry):
            run, d0, d1 = carry
            c = c2 * 2
            m0 = cnt[fbase + c, :]
            m1 = cnt[fbase + c + 1, :]
            cum0 = pfx + run
            run1 = run + m0
            cum1 = pfx + run1
            part0 = jnp.minimum(jnp.maximum(K - cum0, 0.0), m0)
            part1 = jnp.minimum(jnp.maximum(K - cum1, 0.0), m1)
            d0 = d0 + part0 * emid[c, :]
            d1 = d1 + part1 * emid[c + 1, :]
            return run1 + m1, d0, d1

        _, d0, d1 = lax.fori_loop(
            0, NPOS // 2, acc_d, (zeros, zeros, zeros), unroll=4
        )
        dacc[r, :] = d0 + d1
        return 0

    lax.fori_loop(0, RPW, scan_row, 0)
    pltpu.sync_copy(dacc, dpart_hbm.at[pl.ds(base, RPW)])


_sc_select = functools.partial(
    pl.kernel,
    out_type=jax.ShapeDtypeStruct((N, LANES), jnp.float32),
    mesh=plsc.VectorSubcoreMesh(core_axis_name="c", subcore_axis_name="s"),
    compiler_params=pltpu.CompilerParams(needs_layout_passes=False, use_tc_tiling_on_sc=False),
    scratch_types=[
        pltpu.VMEM((RPW, N), jnp.float32),
        pltpu.VMEM((RPW,), jnp.float32),
        pltpu.VMEM((RPW * NPOS, LANES), jnp.float32),
        pltpu.VMEM((NPOS, LANES), jnp.float32),
        pltpu.VMEM((RPW, LANES), jnp.float32),
        pltpu.SemaphoreType.DMA,
    ],
)(_sc_body)


def _final_kernel(dpart_ref, pos_ref, out_ref):
    d = jnp.sum(dpart_ref[...], axis=1, keepdims=True)
    loss = jnp.sum(jnp.log(d) - 2.0 * pos_ref[...]) * (1.0 / N)
    out_ref[...] = jnp.full((1, 1), loss, dtype=jnp.float32)


@jax.jit
def kernel(emb_cat):
    u, inv, pos = pl.pallas_call(
        _prep_kernel,
        out_shape=[
            jax.ShapeDtypeStruct((N, N), jnp.float32),
            jax.ShapeDtypeStruct((N, 1), jnp.float32),
            jax.ShapeDtypeStruct((N, 1), jnp.float32),
        ],
    )(emb_cat)
    dpart = _sc_select(u, inv.reshape(N))
    out = pl.pallas_call(
        _final_kernel,
        out_shape=jax.ShapeDtypeStruct((1, 1), jnp.float32),
    )(dpart, pos)
    return out[0, 0]


# speedup vs baseline: 3.4314x; 3.4314x over previous
import functools

import jax
import jax.numpy as jnp
from jax.experimental import pallas as pl

N = 1024
K = 102
T_ITERS = 11


def _loss_kernel(emb_ref, out_ref):
    e = emb_ref[...]
    eb = e.astype(jnp.bfloat16)
    g = jax.lax.dot_general(
        eb, eb, (((1,), (1,)), ((), ())), preferred_element_type=jnp.float32
    )

    rows = jax.lax.broadcasted_iota(jnp.int32, (N, N), 0)
    cols = jax.lax.broadcasted_iota(jnp.int32, (N, N), 1)
    diag = jnp.sum(jnp.where(rows == cols, g, 0.0), axis=1, keepdims=True)
    nrm = jnp.sqrt(diag)
    inv = 1.0 / jnp.maximum(nrm, 1e-12)

    u = g * inv.reshape(1, N)

    lo0 = -1.5 * nrm
    hi0 = 1.5 * nrm

    def body(_, carry):
        lo, hi = carry
        mid = 0.5 * (lo + hi)
        cnt = jnp.sum((u <= mid).astype(jnp.float32), axis=1, keepdims=True)
        take = cnt >= K
        return jnp.where(take, lo, mid), jnp.where(take, mid, hi)

    lo, hi = jax.lax.fori_loop(0, T_ITERS, body, (lo0, hi0))

    below = u <= lo
    cnt_below = jnp.sum(below.astype(jnp.float32), axis=1, keepdims=True)
    expm = jnp.exp((2.0 * inv) * u)
    sum_below = jnp.sum(jnp.where(below, expm, 0.0), axis=1, keepdims=True)
    denom = sum_below + (K - cnt_below) * jnp.exp(inv * (lo + hi))

    shift = jnp.where(rows < N // 2, rows + N // 2, rows - N // 2)
    pos = inv * jnp.sum(jnp.where(cols == shift, u, 0.0), axis=1, keepdims=True)

    loss_rows = jnp.log(denom) - 2.0 * pos
    loss = jnp.sum(loss_rows) * (1.0 / N)
    out_ref[...] = jnp.full((1, 1), loss, dtype=jnp.float32)


@jax.jit
def kernel(emb_cat):
    out = pl.pallas_call(
        _loss_kernel,
        out_shape=jax.ShapeDtypeStruct((1, 1), jnp.float32),
    )(emb_cat)
    return out[0, 0]


# speedup vs baseline: 3.6097x; 1.0520x over previous
import functools

import jax
import jax.numpy as jnp
from jax.experimental import pallas as pl

N = 1024
K = 102
T_ITERS = 10


def _loss_kernel(emb_ref, out_ref):
    e = emb_ref[...]
    eb = e.astype(jnp.bfloat16)
    g = jax.lax.dot_general(
        eb, eb, (((1,), (1,)), ((), ())), preferred_element_type=jnp.float32
    )

    rows = jax.lax.broadcasted_iota(jnp.int32, (N, N), 0)
    cols = jax.lax.broadcasted_iota(jnp.int32, (N, N), 1)
    shift = jnp.where(rows < N // 2, rows + N // 2, rows - N // 2)
    diag = jnp.sum(jnp.where(rows == cols, g, 0.0), axis=1, keepdims=True)
    gpair = jnp.sum(jnp.where(cols == shift, g, 0.0), axis=1, keepdims=True)
    nrm = jnp.sqrt(diag)
    inv = 1.0 / jnp.maximum(nrm, 1e-12)
    inv_pair = jnp.concatenate([inv[N // 2 :], inv[: N // 2]], axis=0)

    u = g * inv.reshape(1, N)

    lo0 = -nrm
    hi0 = nrm

    def body(_, carry):
        lo, hi = carry
        mid = 0.5 * (lo + hi)
        cnt = jnp.sum((u <= mid).astype(jnp.float32), axis=1, keepdims=True)
        take = cnt >= K
        return jnp.where(take, lo, mid), jnp.where(take, mid, hi)

    lo, hi = jax.lax.fori_loop(0, T_ITERS, body, (lo0, hi0))

    below = u <= lo
    cnt_below = jnp.sum(below.astype(jnp.float32), axis=1, keepdims=True)
    expm = jnp.exp((2.0 * inv) * u)
    sum_below = jnp.sum(jnp.where(below, expm, 0.0), axis=1, keepdims=True)
    denom = sum_below + (K - cnt_below) * jnp.exp(inv * (lo + hi))

    pos = inv * inv_pair * gpair

    loss_rows = jnp.log(denom) - 2.0 * pos
    loss = jnp.sum(loss_rows) * (1.0 / N)
    out_ref[...] = jnp.full((1, 1), loss, dtype=jnp.float32)


@jax.jit
def kernel(emb_cat):
    out = pl.pallas_call(
        _loss_kernel,
        out_shape=jax.ShapeDtypeStruct((1, 1), jnp.float32),
    )(emb_cat)
    return out[0, 0]
